# baseline (speedup 1.0000x reference)
import jax
import jax.numpy as jnp
from jax.experimental import pallas as pl


def _bmu_body(x_ref, w_ref, idx_ref):
    xb = x_ref[...]
    wb = w_ref[...]
    cross = jax.lax.dot_general(
        xb, wb, (((1,), (1,)), ((), ())),
        preferred_element_type=jnp.float32)
    x_sq = jnp.sum(xb * xb, axis=1, keepdims=True)
    w_sq = jnp.sum(wb * wb, axis=1)[None, :]
    dist = x_sq - 2.0 * cross + w_sq
    idx_ref[...] = jnp.argmin(dist, axis=1).astype(jnp.int32)


def kernel(x, weights):
    H, W, D = weights.shape
    B = x.shape[0]
    N = H * W
    wf = weights.reshape(N, D)

    BB = 256
    nb = B // BB

    indices = pl.pallas_call(
        _bmu_body,
        grid=(nb,),
        in_specs=[
            pl.BlockSpec((BB, D), lambda i: (i, 0)),
            pl.BlockSpec((N, D), lambda i: (0, 0)),
        ],
        out_specs=pl.BlockSpec((BB,), lambda i: (i,)),
        out_shape=jax.ShapeDtypeStruct((B,), jnp.int32),
    )(x, wf)

    coords = jnp.stack([indices // W, indices % W], axis=1)
    return coords, indices
